# baseline (speedup 1.0000x reference)
import functools

import jax
import jax.numpy as jnp
from jax import lax
from jax.experimental import pallas as pl
from jax.experimental.pallas import tpu as pltpu
from jax.experimental.pallas import tpu_sc as plsc


def kernel(token_id, table):
    B = token_id.shape[0]
    V, D = table.shape
    table1 = table.reshape(V * D)
    info = plsc.get_sparse_core_info()
    NC, NS, L = info.num_cores, info.num_subcores, info.num_lanes
    NW = NC * NS
    assert B % (8 * NW) == 0
    b_per_w = B // NW
    n_idx = b_per_w * D
    CHUNK = 128
    mesh = plsc.VectorSubcoreMesh(core_axis_name="c", subcore_axis_name="s")

    @functools.partial(
        pl.kernel,
        mesh=mesh,
        out_type=jax.ShapeDtypeStruct((B * D,), jnp.float32),
        scratch_types=[
            pltpu.VMEM((b_per_w,), jnp.int32),
            pltpu.VMEM((n_idx,), jnp.int32),
            pltpu.VMEM((n_idx,), jnp.float32),
            pltpu.SemaphoreType.DMA,
        ],
        compiler_params=pltpu.CompilerParams(
            use_tc_tiling_on_sc=False, needs_layout_passes=False
        ),
    )
    def gather_kernel(idx_hbm, t1_hbm, out_hbm, idx_v, gidx_v, out_v, sem):
        wid = lax.axis_index("s") * NC + lax.axis_index("c")
        base = wid * b_per_w
        lanes = lax.iota(jnp.int32, L)

        pltpu.sync_copy(idx_hbm.at[pl.ds(base, b_per_w)], idx_v)

        def build(g, carry):
            m = g * L + lanes
            toks = plsc.load_gather(idx_v, [m // D])
            gidx_v[pl.ds(g * L, L)] = toks * D + (m % D)
            return carry

        lax.fori_loop(0, n_idx // L, build, None)

        LAG = 8

        def start_chunk(k, carry):
            pltpu.make_async_copy(
                t1_hbm.at[gidx_v.at[pl.ds(k * CHUNK, CHUNK)]],
                out_v.at[pl.ds(k * CHUNK, CHUNK)],
                sem,
            ).start()
            return carry

        def drain_chunk(k, carry):
            pltpu.make_async_copy(
                t1_hbm.at[gidx_v.at[pl.ds(0, CHUNK)]],
                out_v.at[pl.ds(0, CHUNK)],
                sem,
            ).wait()
            return carry

        def run_chunk(k, carry):
            start_chunk(k, None)
            lax.cond(
                k >= LAG,
                lambda: lax.fori_loop(0, 1, drain_chunk, None),
                lambda: None,
            )
            return carry

        lax.fori_loop(0, n_idx // CHUNK, run_chunk, None)
        lax.fori_loop(0, LAG, drain_chunk, None)

        pltpu.sync_copy(out_v, out_hbm.at[pl.ds(base * D, n_idx)])

    out_flat = gather_kernel(token_id.astype(jnp.int32), table1)
    return out_flat.reshape(B, D)


# speedup vs baseline: 1.7625x; 1.7625x over previous
import functools

import jax
import jax.numpy as jnp
from jax import lax
from jax.experimental import pallas as pl
from jax.experimental.pallas import tpu as pltpu
from jax.experimental.pallas import tpu_sc as plsc


def kernel(token_id, table):
    B = token_id.shape[0]
    V, D = table.shape
    info = plsc.get_sparse_core_info()
    NC, NS, L = info.num_cores, info.num_subcores, info.num_lanes
    NW = NC * NS
    assert B % (8 * NW) == 0
    b_per_w = B // NW
    LAG = 7
    mesh = plsc.VectorSubcoreMesh(core_axis_name="c", subcore_axis_name="s")

    @functools.partial(
        pl.kernel,
        mesh=mesh,
        out_type=jax.ShapeDtypeStruct((B, D), jnp.float32),
        scratch_types=[
            pltpu.VMEM((b_per_w,), jnp.int32),
            pltpu.VMEM((b_per_w, D), jnp.float32),
            pltpu.SemaphoreType.DMA,
        ],
    )
    def gather_kernel(idx_hbm, table_hbm, out_hbm, idx_v, rows_v, sem):
        wid = lax.axis_index("s") * NC + lax.axis_index("c")
        base = wid * b_per_w
        pltpu.sync_copy(idx_hbm.at[pl.ds(base, b_per_w)], idx_v)

        def drain_one(i, carry):
            pltpu.make_async_copy(
                table_hbm.at[pl.ds(0, 1)],
                rows_v.at[pl.ds(0, 1)],
                sem,
            ).wait()
            return carry

        def group(g, carry):
            toks = idx_v[pl.ds(g * L, L)]
            for j in range(L):
                pltpu.make_async_copy(
                    table_hbm.at[pl.ds(toks[j], 1)],
                    rows_v.at[pl.ds(g * L + j, 1)],
                    sem,
                ).start()
            lax.cond(
                g >= LAG,
                lambda: lax.fori_loop(0, L, drain_one, None),
                lambda: None,
            )
            return carry

        lax.fori_loop(0, b_per_w // L, group, None)
        lax.fori_loop(0, LAG * L, drain_one, None)

        pltpu.sync_copy(rows_v, out_hbm.at[pl.ds(base, b_per_w)])

    return gather_kernel(token_id.astype(jnp.int32), table)
